# baseline (speedup 1.0000x reference)
import jax
import jax.numpy as jnp
from jax.experimental import pallas as pl
from jax.experimental.pallas import tpu as pltpu


def _gather_copy_kernel(x_ref, o_ref, sem):
    c0 = pltpu.make_async_copy(x_ref.at[pl.ds(0, 4)], o_ref.at[pl.ds(0, 4)],
                               sem.at[0])
    c1 = pltpu.make_async_copy(x_ref.at[pl.ds(7, 4)], o_ref.at[pl.ds(4, 4)],
                               sem.at[1])
    c0.start()
    c1.start()
    c0.wait()
    c1.wait()


def kernel(x):
    xf = x.reshape(14, 2048, 2048)
    return pl.pallas_call(
        _gather_copy_kernel,
        out_shape=jax.ShapeDtypeStruct((8, 2048, 2048), x.dtype),
        in_specs=[pl.BlockSpec(memory_space=pltpu.MemorySpace.HBM)],
        out_specs=pl.BlockSpec(memory_space=pltpu.MemorySpace.HBM),
        scratch_shapes=[pltpu.SemaphoreType.DMA((2,))],
    )(xf)
